# baseline (speedup 1.0000x reference)
import functools

import jax
import jax.numpy as jnp
from jax import lax
from jax.experimental import pallas as pl
from jax.experimental.pallas import tpu as pltpu
from jax.experimental.pallas import tpu_sc as plsc

B, N, K, D = 8, 10000, 5000, 256
NWPB = 4
RNG = N // NWPB
CH = 64
NCHMAX = (RNG + CH - 1) // CH
NKCH = (K + 15) // 16
NRCH = (RNG + 15) // 16


def _make_unpool():
  mesh = plsc.VectorSubcoreMesh(core_axis_name="c", subcore_axis_name="s")

  @functools.partial(
      pl.kernel,
      mesh=mesh,
      compiler_params=pltpu.CompilerParams(needs_layout_passes=False),
      out_type=jax.ShapeDtypeStruct((B, N, D), jnp.float32),
      scratch_types=[
          pltpu.VMEM((K,), jnp.int32),
          pltpu.VMEM((N,), jnp.int32),
          pltpu.VMEM((NCHMAX, CH), jnp.int32),
          pltpu.VMEM((NCHMAX, CH), jnp.int32),
          pltpu.VMEM((NCHMAX, CH), jnp.int32),
          pltpu.VMEM((CH, D), jnp.float32),
          pltpu.VMEM((3 * CH, D), jnp.float32),
          pltpu.SemaphoreType.DMA,
          pltpu.SemaphoreType.DMA,
          pltpu.SemaphoreType.DMA,
      ],
  )
  def unpool(h_hbm, idx_hbm, out_hbm, idx_v, winmap, src2d, dst2d, zdst2d,
             zerobuf, rowbufs, sem_g, sem_s, sem_z):
    wid = lax.axis_index("c") * 16 + lax.axis_index("s")
    b = wid // NWPB
    n0 = (wid % NWPB) * RNG
    iota = lax.iota(jnp.int32, 16)

    pltpu.make_async_copy(idx_hbm.at[b], idx_v, sem_g).start()

    z16 = jnp.zeros((16,), jnp.float32)
    neg1 = jnp.full((16,), -1, jnp.int32)

    def initbuf(r, _):
      for j in range(D // 16):
        zerobuf[r, pl.ds(j * 16, 16)] = z16
      return 0
    lax.fori_loop(0, CH, initbuf, 0)

    def initmap(i, _):
      winmap[pl.ds(i * 16, 16)] = neg1
      return 0
    lax.fori_loop(0, N // 16, initmap, 0)

    pltpu.make_async_copy(idx_hbm.at[b], idx_v, sem_g).wait()

    def p1(i, _):
      for j in range(2):
        base = jnp.minimum(i * 32 + j * 16, K - 16)
        v16 = idx_v[pl.ds(base, 16)]
        k16 = base + iota
        plsc.store_scatter(winmap, [v16], k16, mask=k16 >= 0)
      return 0
    lax.fori_loop(0, (K + 31) // 32, p1, 0)

    h_b = h_hbm.at[b]
    out_b = out_hbm.at[b]

    def buf(c):
      return rowbufs.at[pl.ds((c % 3) * CH, CH)]

    def p2unit(u, cur):
      wcv, zf, gf = cur
      for j in range(CH // 16):
        total = u * CH + j * 16
        base = n0 + jnp.minimum(total, RNG - 16)
        w16 = winmap[pl.ds(base, 16)]
        n16 = base + iota
        m = w16 >= 0
        mi = m.astype(jnp.int32)
        cums = jnp.cumsum(mi)
        cnt = plsc.all_reduce_population_count(m)
        pos = wcv + cums - mi
        plsc.store_scatter(src2d, [pos >> 6, pos & 63], w16, mask=m)
        plsc.store_scatter(dst2d, [pos >> 6, pos & 63], n16, mask=m)
        zpos = (total - wcv) + (iota + 1 - cums) - (1 - mi)
        plsc.store_scatter(zdst2d, [zpos >> 6, zpos & 63], n16,
                           mask=jnp.logical_not(m))
        wcv = wcv + cnt
      wcs = wcv[0]
      zcs = (u + 1) * CH - wcs
      zready = zf < (zcs // CH)
      @pl.when(zready)
      def _():
        pltpu.make_async_copy(zerobuf, out_b.at[zdst2d.at[zf]], sem_z).start()
      gready = gf < jnp.minimum(wcs // CH, 3)
      @pl.when(gready)
      def _():
        pltpu.make_async_copy(h_b.at[src2d.at[gf]], buf(gf), sem_g).start()
      return (wcv, zf + zready.astype(jnp.int32),
              gf + gready.astype(jnp.int32))
    zscal = jnp.zeros((), jnp.int32)
    wcv, zf, gf = lax.fori_loop(
        0, NCHMAX, p2unit, (jnp.zeros((16,), jnp.int32), zscal, zscal))
    wc = wcv[0]
    zc = NCHMAX * CH - wc

    nwch = wc // CH
    nzch = zc // CH

    def zfire(c, _):
      @pl.when((c >= zf) & (c < nzch))
      def _():
        pltpu.make_async_copy(zerobuf, out_b.at[zdst2d.at[c]], sem_z).start()
      return 0
    lax.fori_loop(0, NCHMAX, zfire, 0)

    @pl.when((nwch > 0) & (gf < 1))
    def _():
      pltpu.make_async_copy(h_b.at[src2d.at[0]], buf(0), sem_g).start()
    @pl.when((nwch > 1) & (gf < 2))
    def _():
      pltpu.make_async_copy(h_b.at[src2d.at[1]], buf(1), sem_g).start()

    def wloop(c, _):
      @pl.when(c < nwch)
      def _():
        bc = buf(c)
        pltpu.make_async_copy(h_b.at[src2d.at[c]], bc, sem_g).wait()
        pltpu.make_async_copy(bc, out_b.at[dst2d.at[c]], sem_s).start()
        @pl.when((c + 2 < nwch) & (c + 2 >= gf))
        def _():
          @pl.when(c >= 1)
          def _():
            pltpu.make_async_copy(buf(c + 2), out_b.at[dst2d.at[c]],
                                  sem_s).wait()
          pltpu.make_async_copy(h_b.at[src2d.at[c + 2]], buf(c + 2),
                                sem_g).start()
      return 0
    lax.fori_loop(0, NCHMAX, wloop, 0)

    for j in range(3):
      @pl.when(nwch > j)
      def _(j=j):
        pltpu.make_async_copy(buf(j), out_b.at[dst2d.at[j]], sem_s).wait()

    def zdrain(c, _):
      @pl.when(c < nzch)
      def _():
        pltpu.make_async_copy(zerobuf, out_b.at[zdst2d.at[c]], sem_z).wait()
      return 0
    lax.fori_loop(0, NCHMAX, zdrain, 0)

    def tail_idx(list2d, row, t, g):
      cmin = jnp.maximum(jnp.minimum(g * 16, t - 16), 0)
      raw = list2d[row, pl.ds(cmin, 16)]
      sel = (cmin + iota) < t
      return jnp.where(sel, raw, jnp.full((16,), 0, jnp.int32) + raw[0])

    wrow, wt = wc // CH, wc % CH
    rb16 = rowbufs.at[pl.ds(0, 16)]
    for g in range(CH // 16):
      @pl.when(g * 16 < wt)
      def _(g=g):
        sv = tail_idx(src2d, wrow, wt, g)
        dv = tail_idx(dst2d, wrow, wt, g)
        pltpu.async_copy(h_b.at[sv], rb16, sem_g).wait()
        pltpu.async_copy(rb16, out_b.at[dv], sem_s).wait()

    zrow, zt = zc // CH, zc % CH
    zb16 = zerobuf.at[pl.ds(0, 16)]
    for g in range(CH // 16):
      @pl.when(g * 16 < zt)
      def _(g=g):
        zv = tail_idx(zdst2d, zrow, zt, g)
        pltpu.async_copy(zb16, out_b.at[zv], sem_s).wait()

  return unpool


_unpool = _make_unpool()


def kernel(g, h, pre_h, idx):
  del pre_h
  new_h = _unpool(h, idx.astype(jnp.int32))
  return (g, new_h)


# speedup vs baseline: 1.0384x; 1.0384x over previous
import functools

import jax
import jax.numpy as jnp
from jax import lax
from jax.experimental import pallas as pl
from jax.experimental.pallas import tpu as pltpu
from jax.experimental.pallas import tpu_sc as plsc

B, N, K, D = 8, 10000, 5000, 256
NWPB = 4
RNG = N // NWPB
CH = 64
NCHMAX = (RNG + CH - 1) // CH
NKCH = (K + 15) // 16
NRCH = (RNG + 15) // 16


def _make_unpool():
  mesh = plsc.VectorSubcoreMesh(core_axis_name="c", subcore_axis_name="s")

  @functools.partial(
      pl.kernel,
      mesh=mesh,
      compiler_params=pltpu.CompilerParams(needs_layout_passes=False),
      out_type=jax.ShapeDtypeStruct((B, N, D), jnp.float32),
      scratch_types=[
          pltpu.VMEM((K,), jnp.int32),
          pltpu.VMEM((N,), jnp.int32),
          pltpu.VMEM((NCHMAX, CH), jnp.int32),
          pltpu.VMEM((NCHMAX, CH), jnp.int32),
          pltpu.VMEM((NCHMAX, CH), jnp.int32),
          pltpu.VMEM((CH, D), jnp.float32),
          pltpu.VMEM((3 * CH, D), jnp.float32),
          pltpu.SemaphoreType.DMA,
          pltpu.SemaphoreType.DMA,
          pltpu.SemaphoreType.DMA,
      ],
  )
  def unpool(h_hbm, idx_hbm, out_hbm, idx_v, winmap, src2d, dst2d, zdst2d,
             zerobuf, rowbufs, sem_g, sem_s, sem_z):
    wid = lax.axis_index("c") * 16 + lax.axis_index("s")
    b = wid // NWPB
    n0 = (wid % NWPB) * RNG
    iota = lax.iota(jnp.int32, 16)

    pltpu.make_async_copy(idx_hbm.at[b], idx_v, sem_g).start()

    z16 = jnp.zeros((16,), jnp.float32)
    neg1 = jnp.full((16,), -1, jnp.int32)

    def initbuf(r, _):
      for j in range(D // 16):
        zerobuf[r, pl.ds(j * 16, 16)] = z16
      return 0
    lax.fori_loop(0, CH, initbuf, 0)

    def initmap(i, _):
      for j in range(8):
        base = n0 + jnp.minimum(i * 128 + j * 16, RNG - 16)
        winmap[pl.ds(base, 16)] = neg1
      return 0
    lax.fori_loop(0, (RNG + 127) // 128, initmap, 0)

    pltpu.make_async_copy(idx_hbm.at[b], idx_v, sem_g).wait()

    def p1(i, _):
      for j in range(4):
        base = jnp.minimum(i * 64 + j * 16, K - 16)
        v16 = idx_v[pl.ds(base, 16)]
        k16 = base + iota
        plsc.store_scatter(winmap, [v16], k16, mask=k16 >= 0)
      return 0
    lax.fori_loop(0, (K + 63) // 64, p1, 0)

    h_b = h_hbm.at[b]
    out_b = out_hbm.at[b]

    def buf(c):
      return rowbufs.at[pl.ds((c % 3) * CH, CH)]

    def p2unit(u, cur):
      wcv, zf, gf = cur
      for j in range(CH // 16):
        total = u * CH + j * 16
        base = n0 + jnp.minimum(total, RNG - 16)
        w16 = winmap[pl.ds(base, 16)]
        n16 = base + iota
        m = w16 >= 0
        mi = m.astype(jnp.int32)
        cums = jnp.cumsum(mi)
        cnt = plsc.all_reduce_population_count(m)
        pos = wcv + cums - mi
        plsc.store_scatter(src2d, [pos >> 6, pos & 63], w16, mask=m)
        plsc.store_scatter(dst2d, [pos >> 6, pos & 63], n16, mask=m)
        zpos = (total - wcv) + (iota + 1 - cums) - (1 - mi)
        plsc.store_scatter(zdst2d, [zpos >> 6, zpos & 63], n16,
                           mask=jnp.logical_not(m))
        wcv = wcv + cnt
      wcs = wcv[0]
      zcs = (u + 1) * CH - wcs
      zready = zf < (zcs // CH)
      @pl.when(zready)
      def _():
        pltpu.make_async_copy(zerobuf, out_b.at[zdst2d.at[zf]], sem_z).start()
      gready = gf < jnp.minimum(wcs // CH, 3)
      @pl.when(gready)
      def _():
        pltpu.make_async_copy(h_b.at[src2d.at[gf]], buf(gf), sem_g).start()
      return (wcv, zf + zready.astype(jnp.int32),
              gf + gready.astype(jnp.int32))
    zscal = jnp.zeros((), jnp.int32)
    wcv, zf, gf = lax.fori_loop(
        0, NCHMAX, p2unit, (jnp.zeros((16,), jnp.int32), zscal, zscal))
    wc = wcv[0]
    zc = NCHMAX * CH - wc

    nwch = wc // CH
    nzch = zc // CH

    def zfire(c, _):
      @pl.when((c >= zf) & (c < nzch))
      def _():
        pltpu.make_async_copy(zerobuf, out_b.at[zdst2d.at[c]], sem_z).start()
      return 0
    lax.fori_loop(0, NCHMAX, zfire, 0)

    @pl.when((nwch > 0) & (gf < 1))
    def _():
      pltpu.make_async_copy(h_b.at[src2d.at[0]], buf(0), sem_g).start()
    @pl.when((nwch > 1) & (gf < 2))
    def _():
      pltpu.make_async_copy(h_b.at[src2d.at[1]], buf(1), sem_g).start()

    def wloop(c, _):
      @pl.when(c < nwch)
      def _():
        bc = buf(c)
        pltpu.make_async_copy(h_b.at[src2d.at[c]], bc, sem_g).wait()
        pltpu.make_async_copy(bc, out_b.at[dst2d.at[c]], sem_s).start()
        @pl.when((c + 2 < nwch) & (c + 2 >= gf))
        def _():
          @pl.when(c >= 1)
          def _():
            pltpu.make_async_copy(buf(c + 2), out_b.at[dst2d.at[c]],
                                  sem_s).wait()
          pltpu.make_async_copy(h_b.at[src2d.at[c + 2]], buf(c + 2),
                                sem_g).start()
      return 0
    lax.fori_loop(0, NCHMAX, wloop, 0)

    for j in range(3):
      @pl.when(nwch > j)
      def _(j=j):
        pltpu.make_async_copy(buf(j), out_b.at[dst2d.at[j]], sem_s).wait()

    def zdrain(c, _):
      @pl.when(c < nzch)
      def _():
        pltpu.make_async_copy(zerobuf, out_b.at[zdst2d.at[c]], sem_z).wait()
      return 0
    lax.fori_loop(0, NCHMAX, zdrain, 0)

    def tail_idx(list2d, row, t, g):
      cmin = jnp.maximum(jnp.minimum(g * 16, t - 16), 0)
      raw = list2d[row, pl.ds(cmin, 16)]
      sel = (cmin + iota) < t
      return jnp.where(sel, raw, jnp.full((16,), 0, jnp.int32) + raw[0])

    wrow, wt = wc // CH, wc % CH
    rb16 = rowbufs.at[pl.ds(0, 16)]
    for g in range(CH // 16):
      @pl.when(g * 16 < wt)
      def _(g=g):
        sv = tail_idx(src2d, wrow, wt, g)
        dv = tail_idx(dst2d, wrow, wt, g)
        pltpu.async_copy(h_b.at[sv], rb16, sem_g).wait()
        pltpu.async_copy(rb16, out_b.at[dv], sem_s).wait()

    zrow, zt = zc // CH, zc % CH
    zb16 = zerobuf.at[pl.ds(0, 16)]
    for g in range(CH // 16):
      @pl.when(g * 16 < zt)
      def _(g=g):
        zv = tail_idx(zdst2d, zrow, zt, g)
        pltpu.async_copy(zb16, out_b.at[zv], sem_s).wait()

  return unpool


_unpool = _make_unpool()


def kernel(g, h, pre_h, idx):
  del pre_h
  new_h = _unpool(h, idx.astype(jnp.int32))
  return (g, new_h)
